# baseline (speedup 1.0000x reference)
import functools

import jax
import jax.numpy as jnp
from jax import lax
from jax.experimental import pallas as pl
from jax.experimental.pallas import tpu as pltpu
from jax.experimental.pallas import tpu_sc as plsc

N = 10000
NP = 10240
C = 96
NT = 16
EPAD = 331776
NPT = NP // NT


def _edge_body(cfg, h2, as_t, ad_t, shift, ech_h, zh, zs,
               out_h, out_s,
               as_v, ad_v, ed_v, gidx_v, dstc_v, w_v, rows_v, shv,
               acc_sh, s_sh, gsem, esem):
    H, HC, NCH, e_tot, split32 = cfg
    c = lax.axis_index("c")
    s = lax.axis_index("s")
    if split32:
        tile = s * 2 + c
    else:
        tile = s
    ch0 = tile * NCH
    pltpu.sync_copy(shift, shv)
    shift_vec = shv[...]
    nsl = pl.ds(s * NPT, NPT)
    nr = C // 16

    for hi in range(HC):
        if split32:
            k = 0
            slot = c
        else:
            k = c * HC + hi
            slot = k
        kn = k * N
        pltpu.sync_copy(as_t.at[k], as_v)
        pltpu.sync_copy(ad_t.at[k], ad_v)
        pltpu.sync_copy(zh, acc_sh.at[nsl])
        pltpu.sync_copy(zs, s_sh.at[nsl])
        plsc.subcore_barrier()

        def prep(ch, b):
            for r in range(nr):
                sl = pl.ds(r * 16, 16)
                sv = ed_v[b, 0, sl]
                dv = ed_v[b, 1, sl]
                gidx_v[b, sl] = sv + kn
                dstc_v[b, sl] = dv
                e = plsc.load_gather(as_v, [sv]) + plsc.load_gather(ad_v, [dv])
                e = jnp.maximum(e, 0.2 * e)
                w = jnp.exp(e - shift_vec)
                gid = (ch0 + ch) * C + r * 16 + lax.iota(jnp.int32, 16)
                w_v[b, sl] = jnp.where(gid < e_tot, w, 0.0)
            pltpu.async_copy(h2.at[gidx_v.at[b]], rows_v.at[b], gsem.at[b])

        def process(ch, b):
            @plsc.parallel_loop(0, C, 1, unroll=4)
            def _(e_i):
                wv = plsc.load_gather(
                    w_v.at[b], [jnp.full((16,), e_i, jnp.int32)])
                for r8 in range(8):
                    sl = pl.ds(r8 * 16, 16)
                    rows_v[b, e_i, sl] = rows_v[b, e_i, sl] * wv
            pltpu.sync_copy(rows_v.at[b], acc_sh.at[dstc_v.at[b]], add=True)
            pltpu.sync_copy(w_v.at[b], s_sh.at[dstc_v.at[b]], add=True)

        pltpu.sync_copy(ech_h.at[ch0], ed_v.at[0])
        prep(0, 0)
        pltpu.async_copy(ech_h.at[ch0 + 1], ed_v.at[1], esem.at[1])

        def chunk2(ch2):
            for b in range(2):
                ch = ch2 + b
                nb = 1 - b

                @pl.when(ch + 2 < NCH)
                def _():
                    pltpu.async_copy(ech_h.at[ch0 + ch + 2], ed_v.at[b],
                                     esem.at[b])

                @pl.when(ch + 1 < NCH)
                def _():
                    pltpu.make_async_copy(ech_h.at[ch0 + ch + 1],
                                          ed_v.at[nb], esem.at[nb]).wait()
                    prep(ch + 1, nb)

                pltpu.make_async_copy(h2.at[gidx_v.at[b]], rows_v.at[b],
                                      gsem.at[b]).wait()
                process(ch, b)

        pl.loop(0, NCH, step=2)(chunk2)
        plsc.subcore_barrier()
        pltpu.sync_copy(acc_sh.at[nsl], out_h.at[slot, nsl])
        pltpu.sync_copy(s_sh.at[nsl], out_s.at[slot, nsl])
        plsc.subcore_barrier()


@functools.cache
def _edge_kernel(H, e_tot, split32):
    HC = 1 if split32 else H // 2
    EPT = EPAD // 32 if split32 else EPAD // 16
    NCH = EPT // C
    S = 2 if split32 else H
    mesh = plsc.VectorSubcoreMesh(core_axis_name="c", subcore_axis_name="s")
    body = functools.partial(_edge_body, (H, HC, NCH, e_tot, split32))
    return pl.kernel(
        body,
        out_type=(jax.ShapeDtypeStruct((S, NP, 128), jnp.float32),
                  jax.ShapeDtypeStruct((S, NP), jnp.float32)),
        mesh=mesh,
        compiler_params=pltpu.CompilerParams(needs_layout_passes=False),
        scratch_types=[
            pltpu.VMEM((N,), jnp.float32),
            pltpu.VMEM((N,), jnp.float32),
            pltpu.VMEM((2, 2, C), jnp.int32),
            pltpu.VMEM((2, C), jnp.int32),
            pltpu.VMEM((2, C), jnp.int32),
            pltpu.VMEM((2, C), jnp.float32),
            pltpu.VMEM((2, C, 128), jnp.float32),
            pltpu.VMEM((16,), jnp.float32),
            pltpu.VMEM_SHARED((NP, 128), jnp.float32),
            pltpu.VMEM_SHARED((NP,), jnp.float32),
            pltpu.SemaphoreType.DMA((2,)),
            pltpu.SemaphoreType.DMA((2,)),
        ],
    )


def _mmattn_body(x_ref, w_ref, as_ref, ad_ref,
                 h_ref, asn_ref, adn_ref, mx_ref, mscr):
    i = pl.program_id(0)
    h = jnp.dot(x_ref[...], w_ref[...], preferred_element_type=jnp.float32)
    h_ref[...] = h
    blk = h.shape[0]
    H = as_ref.shape[0]
    hh = h.reshape(blk, H, 128)
    asn = jnp.sum(hh * as_ref[...][None, :, :], axis=-1)
    adn = jnp.sum(hh * ad_ref[...][None, :, :], axis=-1)
    asn_ref[...] = asn
    adn_ref[...] = adn
    bs = jnp.max(asn)
    bd = jnp.max(adn)

    @pl.when(i == 0)
    def _():
        mscr[0] = bs
        mscr[1] = bd

    @pl.when(i > 0)
    def _():
        mscr[0] = jnp.maximum(mscr[0], bs)
        mscr[1] = jnp.maximum(mscr[1], bd)

    @pl.when(i == pl.num_programs(0) - 1)
    def _():
        mx_ref[0] = mscr[0]
        mx_ref[1] = mscr[1]


def _mm_attn(x, W, a_s, a_d, H):
    n, kdim = x.shape
    m = W.shape[1]
    blk = 1000
    return pl.pallas_call(
        _mmattn_body,
        grid=(n // blk,),
        in_specs=[
            pl.BlockSpec((blk, kdim), lambda i: (i, 0)),
            pl.BlockSpec((kdim, m), lambda i: (0, 0)),
            pl.BlockSpec((H, 128), lambda i: (0, 0)),
            pl.BlockSpec((H, 128), lambda i: (0, 0)),
        ],
        out_specs=[
            pl.BlockSpec((blk, m), lambda i: (i, 0)),
            pl.BlockSpec((blk, H), lambda i: (i, 0)),
            pl.BlockSpec((blk, H), lambda i: (i, 0)),
            pl.BlockSpec(memory_space=pltpu.SMEM),
        ],
        out_shape=[
            jax.ShapeDtypeStruct((n, m), jnp.float32),
            jax.ShapeDtypeStruct((n, H), jnp.float32),
            jax.ShapeDtypeStruct((n, H), jnp.float32),
            jax.ShapeDtypeStruct((2,), jnp.float32),
        ],
        scratch_shapes=[pltpu.SMEM((2,), jnp.float32)],
    )(x, W, a_s, a_d)


def _norm_body(x_ref, s_ref, b_ref, o_ref):
    blk = x_ref.shape[0]
    H = s_ref.shape[1]
    xs = x_ref[...].reshape(blk, H, 128) / s_ref[...][:, :, None]
    v = xs.reshape(blk, H * 128) + b_ref[...]
    o_ref[...] = jnp.where(v > 0, v, jnp.exp(v) - 1.0)


def _norm_elu(xh, s, b):
    n, m = xh.shape
    H = s.shape[1]
    blk = 1000
    return pl.pallas_call(
        _norm_body,
        grid=(n // blk,),
        in_specs=[pl.BlockSpec((blk, m), lambda i: (i, 0)),
                  pl.BlockSpec((blk, H), lambda i: (i, 0)),
                  pl.BlockSpec((1, m), lambda i: (0, 0))],
        out_specs=pl.BlockSpec((blk, m), lambda i: (i, 0)),
        out_shape=jax.ShapeDtypeStruct((n, m), jnp.float32),
    )(xh, s, b.reshape(1, m))


def _norm3_body(x_ref, s_ref, b_ref, o_ref):
    num = x_ref[:, :128] + x_ref[:, 128:]
    den = s_ref[:, 0:1] + s_ref[:, 1:2]
    v = num / den + b_ref[...]
    o_ref[...] = jnp.where(v > 0, v, jnp.exp(v) - 1.0)


def _norm3_elu(xh2, s2, b):
    n = xh2.shape[0]
    blk = 1000
    return pl.pallas_call(
        _norm3_body,
        grid=(n // blk,),
        in_specs=[pl.BlockSpec((blk, 256), lambda i: (i, 0)),
                  pl.BlockSpec((blk, 2), lambda i: (i, 0)),
                  pl.BlockSpec((1, 128), lambda i: (0, 0))],
        out_specs=pl.BlockSpec((blk, 128), lambda i: (i, 0)),
        out_shape=jax.ShapeDtypeStruct((n, 128), jnp.float32),
    )(xh2, s2, b.reshape(1, 128))


def kernel(x, edge_index, batch, params):
    p = params
    n_nodes = x.shape[0]
    e_in = edge_index.shape[1]
    e_tot = e_in + n_nodes
    loop = jnp.arange(n_nodes, dtype=jnp.int32)
    src = jnp.concatenate([edge_index[0].astype(jnp.int32), loop,
                           jnp.zeros((EPAD - e_tot,), jnp.int32)])
    dst = jnp.concatenate([edge_index[1].astype(jnp.int32), loop,
                           jnp.zeros((EPAD - e_tot,), jnp.int32)])
    ech = jnp.stack([src.reshape(EPAD // C, C),
                     dst.reshape(EPAD // C, C)], axis=1)
    zh = jnp.zeros((NPT, 128), jnp.float32)
    zs = jnp.zeros((NPT,), jnp.float32)

    def gat(xin, W, a_s, a_d, b, H):
        h, asn, adn, mx = _mm_attn(xin, W, a_s, a_d, H)
        m = mx[0] + mx[1]
        shift = jnp.full((16,), jnp.maximum(m, 0.2 * m), jnp.float32)
        hh = h.reshape(n_nodes, H, 128)
        h2 = hh.transpose(1, 0, 2).reshape(H * n_nodes, 128)
        split32 = (H == 1)
        out_h, out_s = _edge_kernel(H, e_tot, split32)(
            h2, asn.T, adn.T, shift, ech, zh, zs)
        out_h = out_h[:, :n_nodes]
        out_s = out_s[:, :n_nodes]
        if split32:
            xh2 = out_h.transpose(1, 0, 2).reshape(n_nodes, 256)
            return _norm3_elu(xh2, out_s.T, b)
        xh = out_h.transpose(1, 0, 2).reshape(n_nodes, H * 128)
        return _norm_elu(xh, out_s.T, b)

    h1 = gat(x, p['W1'], p['as1'], p['ad1'], p['b1'], 8)
    h2 = gat(h1, p['W2'], p['as2'], p['ad2'], p['b2'], 4)
    h3 = gat(h2, p['W3'], p['as3'], p['ad3'], p['b3'], 1)
    sm = jax.ops.segment_sum(h3, batch, num_segments=64)
    cnt = jax.ops.segment_sum(jnp.ones((n_nodes,), h3.dtype), batch,
                              num_segments=64)
    g = sm / jnp.maximum(cnt, 1.0)[:, None]
    outs = []
    for i in range(8):
        hp = p['slots'][str(i)]
        hmlp = jnp.maximum(g @ hp['w1'] + hp['b1'], 0.0)
        if 'w3' in hp:
            hmlp = jnp.maximum(hmlp @ hp['w2'] + hp['b2'], 0.0)
            hmlp = hmlp @ hp['w3'] + hp['b3']
        else:
            hmlp = hmlp @ hp['w2'] + hp['b2']
        outs.append(hmlp)
    return tuple(outs)


# speedup vs baseline: 1.0629x; 1.0629x over previous
import functools

import jax
import jax.numpy as jnp
from jax import lax
from jax.experimental import pallas as pl
from jax.experimental.pallas import tpu as pltpu
from jax.experimental.pallas import tpu_sc as plsc

N = 10000
NP = 10240
C = 96
NT = 16
EPAD = 331776
NPT = NP // NT


def _edge_body(cfg, h2, as_t, ad_t, shift, ech_h, zh, zs,
               out_h, out_s,
               as_v, ad_v, ed_v, gidx_v, dstc_v, w_v, rows_v, shv,
               acc_sh, s_sh, gsem, esem):
    H, HC, NCH, e_tot, split32 = cfg
    c = lax.axis_index("c")
    s = lax.axis_index("s")
    if split32:
        tile = s * 2 + c
    else:
        tile = s
    ch0 = tile * NCH
    pltpu.sync_copy(shift, shv)
    shift_vec = shv[...]
    nsl = pl.ds(s * NPT, NPT)
    nr = C // 16

    for hi in range(HC):
        if split32:
            k = 0
            slot = c
        else:
            k = c * HC + hi
            slot = k
        kn = k * N
        pltpu.sync_copy(as_t.at[k], as_v)
        pltpu.sync_copy(ad_t.at[k], ad_v)
        pltpu.sync_copy(zh, acc_sh.at[nsl])
        pltpu.sync_copy(zs, s_sh.at[nsl])
        plsc.subcore_barrier()

        def prep(ch, b):
            for r in range(nr):
                sl = pl.ds(r * 16, 16)
                sv = ed_v[b, 0, sl]
                dv = ed_v[b, 1, sl]
                gidx_v[b, sl] = sv + kn
                dstc_v[b, sl] = dv
                e = plsc.load_gather(as_v, [sv]) + plsc.load_gather(ad_v, [dv])
                e = jnp.maximum(e, 0.2 * e)
                w = jnp.exp(e - shift_vec)
                gid = (ch0 + ch) * C + r * 16 + lax.iota(jnp.int32, 16)
                w_v[b, sl] = jnp.where(gid < e_tot, w, 0.0)
            pltpu.async_copy(h2.at[gidx_v.at[b]], rows_v.at[b], gsem.at[b])

        def process(ch, b):
            @plsc.parallel_loop(0, C, 1, unroll=4)
            def _(e_i):
                wv = plsc.load_gather(
                    w_v.at[b], [jnp.full((16,), e_i, jnp.int32)])
                for r8 in range(8):
                    sl = pl.ds(r8 * 16, 16)
                    rows_v[b, e_i, sl] = rows_v[b, e_i, sl] * wv
            pltpu.sync_copy(rows_v.at[b], acc_sh.at[dstc_v.at[b]], add=True)
            pltpu.sync_copy(w_v.at[b], s_sh.at[dstc_v.at[b]], add=True)

        pltpu.sync_copy(ech_h.at[ch0], ed_v.at[0])
        prep(0, 0)
        pltpu.async_copy(ech_h.at[ch0 + 1], ed_v.at[1], esem.at[1])

        def chunk2(ch2):
            for b in range(2):
                ch = ch2 + b
                nb = 1 - b

                @pl.when(ch + 2 < NCH)
                def _():
                    pltpu.async_copy(ech_h.at[ch0 + ch + 2], ed_v.at[b],
                                     esem.at[b])

                @pl.when(ch + 1 < NCH)
                def _():
                    pltpu.make_async_copy(ech_h.at[ch0 + ch + 1],
                                          ed_v.at[nb], esem.at[nb]).wait()
                    prep(ch + 1, nb)

                pltpu.make_async_copy(h2.at[gidx_v.at[b]], rows_v.at[b],
                                      gsem.at[b]).wait()
                process(ch, b)

        pl.loop(0, NCH, step=2)(chunk2)
        plsc.subcore_barrier()
        pltpu.sync_copy(acc_sh.at[nsl], out_h.at[slot, nsl])
        pltpu.sync_copy(s_sh.at[nsl], out_s.at[slot, nsl])
        plsc.subcore_barrier()


@functools.cache
def _edge_kernel(H, e_tot, split32):
    HC = 1 if split32 else H // 2
    EPT = EPAD // 32 if split32 else EPAD // 16
    NCH = EPT // C
    S = 2 if split32 else H
    mesh = plsc.VectorSubcoreMesh(core_axis_name="c", subcore_axis_name="s")
    body = functools.partial(_edge_body, (H, HC, NCH, e_tot, split32))
    return pl.kernel(
        body,
        out_type=(jax.ShapeDtypeStruct((S, NP, 128), jnp.float32),
                  jax.ShapeDtypeStruct((S, NP), jnp.float32)),
        mesh=mesh,
        compiler_params=pltpu.CompilerParams(needs_layout_passes=False),
        scratch_types=[
            pltpu.VMEM((N,), jnp.float32),
            pltpu.VMEM((N,), jnp.float32),
            pltpu.VMEM((2, 2, C), jnp.int32),
            pltpu.VMEM((2, C), jnp.int32),
            pltpu.VMEM((2, C), jnp.int32),
            pltpu.VMEM((2, C), jnp.float32),
            pltpu.VMEM((2, C, 128), jnp.float32),
            pltpu.VMEM((16,), jnp.float32),
            pltpu.VMEM_SHARED((NP, 128), jnp.float32),
            pltpu.VMEM_SHARED((NP,), jnp.float32),
            pltpu.SemaphoreType.DMA((2,)),
            pltpu.SemaphoreType.DMA((2,)),
        ],
    )


def _mmattn_body(x_ref, w_ref, as_ref, ad_ref,
                 h_ref, asn_ref, adn_ref, mx_ref, mscr):
    i = pl.program_id(0)
    h = jnp.dot(x_ref[...], w_ref[...], preferred_element_type=jnp.float32)
    h_ref[...] = h
    blk = h.shape[0]
    H = as_ref.shape[0]
    hh = h.reshape(blk, H, 128)
    asn = jnp.sum(hh * as_ref[...][None, :, :], axis=-1)
    adn = jnp.sum(hh * ad_ref[...][None, :, :], axis=-1)
    asn_ref[...] = asn
    adn_ref[...] = adn
    bs = jnp.max(asn)
    bd = jnp.max(adn)

    @pl.when(i == 0)
    def _():
        mscr[0] = bs
        mscr[1] = bd

    @pl.when(i > 0)
    def _():
        mscr[0] = jnp.maximum(mscr[0], bs)
        mscr[1] = jnp.maximum(mscr[1], bd)

    @pl.when(i == pl.num_programs(0) - 1)
    def _():
        mx_ref[0] = mscr[0]
        mx_ref[1] = mscr[1]


def _mm_attn(x, W, a_s, a_d, H):
    n, kdim = x.shape
    m = W.shape[1]
    blk = 1000
    return pl.pallas_call(
        _mmattn_body,
        grid=(n // blk,),
        in_specs=[
            pl.BlockSpec((blk, kdim), lambda i: (i, 0)),
            pl.BlockSpec((kdim, m), lambda i: (0, 0)),
            pl.BlockSpec((H, 128), lambda i: (0, 0)),
            pl.BlockSpec((H, 128), lambda i: (0, 0)),
        ],
        out_specs=[
            pl.BlockSpec((blk, m), lambda i: (i, 0)),
            pl.BlockSpec((blk, H), lambda i: (i, 0)),
            pl.BlockSpec((blk, H), lambda i: (i, 0)),
            pl.BlockSpec(memory_space=pltpu.SMEM),
        ],
        out_shape=[
            jax.ShapeDtypeStruct((n, m), jnp.float32),
            jax.ShapeDtypeStruct((n, H), jnp.float32),
            jax.ShapeDtypeStruct((n, H), jnp.float32),
            jax.ShapeDtypeStruct((2,), jnp.float32),
        ],
        scratch_shapes=[pltpu.SMEM((2,), jnp.float32)],
    )(x, W, a_s, a_d)


def _norm_body(x_ref, s_ref, b_ref, o_ref):
    blk = x_ref.shape[0]
    H = s_ref.shape[1]
    xs = x_ref[...].reshape(blk, H, 128) / s_ref[...][:, :, None]
    v = xs.reshape(blk, H * 128) + b_ref[...]
    o_ref[...] = jnp.where(v > 0, v, jnp.exp(v) - 1.0)


def _norm_elu(xh, s, b):
    n, m = xh.shape
    H = s.shape[1]
    blk = 1000
    return pl.pallas_call(
        _norm_body,
        grid=(n // blk,),
        in_specs=[pl.BlockSpec((blk, m), lambda i: (i, 0)),
                  pl.BlockSpec((blk, H), lambda i: (i, 0)),
                  pl.BlockSpec((1, m), lambda i: (0, 0))],
        out_specs=pl.BlockSpec((blk, m), lambda i: (i, 0)),
        out_shape=jax.ShapeDtypeStruct((n, m), jnp.float32),
    )(xh, s, b.reshape(1, m))


def _norm3_body(x_ref, s_ref, b_ref, o_ref):
    num = x_ref[:, :128] + x_ref[:, 128:]
    den = s_ref[:, 0:1] + s_ref[:, 1:2]
    v = num / den + b_ref[...]
    o_ref[...] = jnp.where(v > 0, v, jnp.exp(v) - 1.0)


def _norm3_elu(xh2, s2, b):
    n = xh2.shape[0]
    blk = 1000
    return pl.pallas_call(
        _norm3_body,
        grid=(n // blk,),
        in_specs=[pl.BlockSpec((blk, 256), lambda i: (i, 0)),
                  pl.BlockSpec((blk, 2), lambda i: (i, 0)),
                  pl.BlockSpec((1, 128), lambda i: (0, 0))],
        out_specs=pl.BlockSpec((blk, 128), lambda i: (i, 0)),
        out_shape=jax.ShapeDtypeStruct((n, 128), jnp.float32),
    )(xh2, s2, b.reshape(1, 128))


def _pool_mlp_body(*refs):
    x_ref, b_ref = refs[0], refs[1]
    wrefs = refs[2:-10]
    outs = refs[-10:-2]
    acc, cnt = refs[-2], refs[-1]
    i = pl.program_id(0)
    oh = (lax.broadcasted_iota(jnp.int32, (64, x_ref.shape[0]), 0)
          == b_ref[0]).astype(jnp.float32)
    part = jax.lax.dot_general(
        oh, x_ref[...], (((1,), (0,)), ((), ())),
        precision=jax.lax.Precision.HIGHEST)
    csum = jnp.sum(oh, axis=1, keepdims=True)

    @pl.when(i == 0)
    def _():
        acc[...] = part
        cnt[...] = csum

    @pl.when(i > 0)
    def _():
        acc[...] = acc[...] + part
        cnt[...] = cnt[...] + csum

    @pl.when(i == pl.num_programs(0) - 1)
    def _():
        g = acc[...] / jnp.maximum(cnt[...], 1.0)
        wi = 0
        for o in outs:
            three = o.shape[1] == 3
            nw = 6 if three else 4
            wr = wrefs[wi:wi + nw]
            wi += nw
            h = jnp.maximum(jnp.dot(g, wr[0][...],
                                    preferred_element_type=jnp.float32)
                            + wr[1][...], 0.0)
            if three:
                h = jnp.maximum(jnp.dot(h, wr[2][...],
                                        preferred_element_type=jnp.float32)
                                + wr[3][...], 0.0)
                o[...] = jnp.dot(h, wr[4][...],
                                 preferred_element_type=jnp.float32) + wr[5][...]
            else:
                o[...] = jnp.dot(h, wr[2][...],
                                 preferred_element_type=jnp.float32) + wr[3][...]


def _pool_mlp(h3, batch, slots):
    n = h3.shape[0]
    blk = 1000
    wlist, wspecs = [], []
    out_shapes, out_specs = [], []
    for i in range(8):
        hp = slots[str(i)]
        names = ('w1', 'b1', 'w2', 'b2') + (('w3', 'b3') if 'w3' in hp else ())
        for nm in names:
            t = hp[nm]
            t2 = t.reshape(1, -1) if t.ndim == 1 else t
            wlist.append(t2)
            wspecs.append(pl.BlockSpec(t2.shape, lambda i: (0, 0)))
        odim = 3 if 'w3' in hp else 2
        out_shapes.append(jax.ShapeDtypeStruct((64, odim), jnp.float32))
        out_specs.append(pl.BlockSpec((64, odim), lambda i: (0, 0)))
    b3d = batch.astype(jnp.int32).reshape(n // blk, 1, blk)
    return pl.pallas_call(
        _pool_mlp_body,
        grid=(n // blk,),
        in_specs=[pl.BlockSpec((blk, 128), lambda i: (i, 0)),
                  pl.BlockSpec((1, 1, blk), lambda i: (i, 0, 0))] + wspecs,
        out_specs=out_specs,
        out_shape=out_shapes,
        scratch_shapes=[pltpu.VMEM((64, 128), jnp.float32),
                        pltpu.VMEM((64, 1), jnp.float32)],
    )(h3, b3d, *wlist)


def kernel(x, edge_index, batch, params):
    p = params
    n_nodes = x.shape[0]
    e_in = edge_index.shape[1]
    e_tot = e_in + n_nodes
    loop = jnp.arange(n_nodes, dtype=jnp.int32)
    src = jnp.concatenate([edge_index[0].astype(jnp.int32), loop,
                           jnp.zeros((EPAD - e_tot,), jnp.int32)])
    dst = jnp.concatenate([edge_index[1].astype(jnp.int32), loop,
                           jnp.zeros((EPAD - e_tot,), jnp.int32)])
    ech = jnp.stack([src.reshape(EPAD // C, C),
                     dst.reshape(EPAD // C, C)], axis=1)
    zh = jnp.zeros((NPT, 128), jnp.float32)
    zs = jnp.zeros((NPT,), jnp.float32)

    def gat(xin, W, a_s, a_d, b, H):
        h, asn, adn, mx = _mm_attn(xin, W, a_s, a_d, H)
        m = mx[0] + mx[1]
        shift = jnp.full((16,), jnp.maximum(m, 0.2 * m), jnp.float32)
        hh = h.reshape(n_nodes, H, 128)
        h2 = hh.transpose(1, 0, 2).reshape(H * n_nodes, 128)
        split32 = (H == 1)
        out_h, out_s = _edge_kernel(H, e_tot, split32)(
            h2, asn.T, adn.T, shift, ech, zh, zs)
        out_h = out_h[:, :n_nodes]
        out_s = out_s[:, :n_nodes]
        if split32:
            xh2 = out_h.transpose(1, 0, 2).reshape(n_nodes, 256)
            return _norm3_elu(xh2, out_s.T, b)
        xh = out_h.transpose(1, 0, 2).reshape(n_nodes, H * 128)
        return _norm_elu(xh, out_s.T, b)

    h1 = gat(x, p['W1'], p['as1'], p['ad1'], p['b1'], 8)
    h2 = gat(h1, p['W2'], p['as2'], p['ad2'], p['b2'], 4)
    h3 = gat(h2, p['W3'], p['as3'], p['ad3'], p['b3'], 1)
    return tuple(_pool_mlp(h3, batch, p['slots']))
